# baseline (device time: 24672 ns/iter reference)
import jax
import jax.numpy as jnp
from jax import lax
from jax.experimental import pallas as pl
from jax.experimental.pallas import tpu as pltpu

N_DEV = 4
B_LOC = 2
H_TOT = 16
H_LOC = 4
H_HALF = 2
SQ = 128
DH = 64
DHH = H_HALF * DH
SCALE = 0.125


def kernel(x, Wq, K_ext, V_ext, Wo):
    my = lax.axis_index("i")
    K_r = K_ext.reshape(K_ext.shape[0], SQ, H_TOT * DH)
    V_r = V_ext.reshape(V_ext.shape[0], SQ, H_TOT * DH)

    def body(x_ref, wq_ref, k_any, v_any, wo_ref, out_hbm,
             x16, acc, k32, v32, snd_wq_cw, snd_wq_ccw, snd_wo_cw, snd_wo_ccw,
             l_wq_cw, l_wo_cw, l_wq_ccw, l_wo_ccw,
             r_wq_cw, r_wo_cw, r_wq_ccw, r_wo_ccw,
             o_wq_cw, o_wo_cw, o_wq_ccw, o_wo_ccw,
             kt, vt, raw_sems, send_sems, recv_sems):
        me = lax.axis_index("i")
        right = lax.rem(me + 1, N_DEV)
        left = lax.rem(me + N_DEV - 1, N_DEV)
        opp = lax.rem(me + 2, N_DEV)

        raw_dmas = [
            pltpu.make_async_copy(
                k_any.at[pl.ds(me * B_LOC, B_LOC)], k32, raw_sems.at[0]
            ),
            pltpu.make_async_copy(
                v_any.at[pl.ds(me * B_LOC, B_LOC)], v32, raw_sems.at[1]
            ),
        ]
        for d in raw_dmas:
            d.start()

        barrier_sem = pltpu.get_barrier_semaphore()
        for nbr in (left, right):
            pl.semaphore_signal(
                barrier_sem, inc=1,
                device_id=(nbr,), device_id_type=pl.DeviceIdType.MESH,
            )
        pl.semaphore_wait(barrier_sem, 2)

        snd_wq_cw[...] = wq_ref[:, :DHH].astype(jnp.bfloat16)
        snd_wq_ccw[...] = wq_ref[:, DHH:].astype(jnp.bfloat16)
        snd_wo_cw[...] = wo_ref[:DHH, :].astype(jnp.bfloat16)
        snd_wo_ccw[...] = wo_ref[DHH:, :].astype(jnp.bfloat16)

        def mk(src, dst, idx, nbr):
            return pltpu.make_async_remote_copy(
                src_ref=src, dst_ref=dst,
                send_sem=send_sems.at[idx], recv_sem=recv_sems.at[idx],
                device_id=(nbr,), device_id_type=pl.DeviceIdType.MESH,
            )

        hop1 = [
            mk(snd_wq_cw, l_wq_cw, 0, right),
            mk(snd_wo_cw, l_wo_cw, 1, right),
            mk(snd_wq_ccw, l_wq_ccw, 2, right),
            mk(snd_wo_ccw, l_wo_ccw, 3, right),
            mk(snd_wq_ccw, r_wq_ccw, 4, left),
            mk(snd_wo_ccw, r_wo_ccw, 5, left),
            mk(snd_wq_cw, r_wq_cw, 6, left),
            mk(snd_wo_cw, r_wo_cw, 7, left),
        ]
        for r in hop1:
            r.start()

        x16[...] = x_ref[...].astype(jnp.bfloat16)

        for d in raw_dmas:
            d.wait()
        for b in range(B_LOC):
            kt[b] = k32[b].astype(jnp.bfloat16)
            vt[b] = v32[b].astype(jnp.bfloat16)

        def attend_half(b, q_half, o, head_off, wo_half):
            col0 = o * H_LOC * DH + head_off * DH
            k_pair = kt[b, :, pl.ds(col0, DHH)]
            v_pair = vt[b, :, pl.ds(col0, DHH)]
            ctx_parts = []
            for hh in range(H_HALF):
                k = k_pair[:, hh * DH:(hh + 1) * DH]
                v = v_pair[:, hh * DH:(hh + 1) * DH]
                q = q_half[:, hh * DH:(hh + 1) * DH]
                s = lax.dot_general(
                    q, k, (((1,), (1,)), ((), ())),
                    preferred_element_type=jnp.float32,
                ) * SCALE
                m = jnp.max(s, axis=-1, keepdims=True)
                w = jnp.exp(s - m)
                w = (w / jnp.sum(w, axis=-1, keepdims=True)).astype(
                    jnp.bfloat16
                )
                ctx_parts.append(
                    jnp.dot(w, v, preferred_element_type=jnp.float32)
                )
            ctx = jnp.concatenate(ctx_parts, axis=1).astype(jnp.bfloat16)
            return jnp.dot(ctx, wo_half, preferred_element_type=jnp.float32)

        def compute_half(origin, head_off, wq_half_ref, wo_half_ref,
                         init=False):
            wq_half = wq_half_ref[...]
            wo_half = wo_half_ref[...]
            for b in range(B_LOC):
                q = jnp.dot(
                    x16[b], wq_half, preferred_element_type=jnp.float32
                ).astype(jnp.bfloat16)
                c = attend_half(b, q, origin, head_off, wo_half)
                if init:
                    acc[b] = c
                else:
                    acc[b] = acc[b] + c

        compute_half(me, 0, snd_wq_cw, snd_wo_cw, init=True)
        compute_half(me, H_HALF, snd_wq_ccw, snd_wo_ccw)

        hop1[0].wait_recv()
        hop1[1].wait_recv()
        hop2 = [
            mk(l_wq_cw, o_wq_cw, 8, right),
            mk(l_wo_cw, o_wo_cw, 9, right),
        ]
        hop1[4].wait_recv()
        hop1[5].wait_recv()
        hop2 += [
            mk(r_wq_ccw, o_wq_ccw, 10, left),
            mk(r_wo_ccw, o_wo_ccw, 11, left),
        ]
        for r in hop2:
            r.start()

        compute_half(left, 0, l_wq_cw, l_wo_cw)
        compute_half(right, H_HALF, r_wq_ccw, r_wo_ccw)

        hop1[2].wait_recv()
        hop1[3].wait_recv()
        compute_half(left, H_HALF, l_wq_ccw, l_wo_ccw)

        hop1[6].wait_recv()
        hop1[7].wait_recv()
        compute_half(right, 0, r_wq_cw, r_wo_cw)

        hop2[0].wait_recv()
        hop2[1].wait_recv()
        compute_half(opp, 0, o_wq_cw, o_wo_cw)

        hop2[2].wait_recv()
        hop2[3].wait_recv()
        compute_half(opp, H_HALF, o_wq_ccw, o_wo_ccw)

        out_dma = pltpu.make_async_copy(acc, out_hbm, raw_sems.at[0])
        out_dma.start()

        for r in hop1 + hop2:
            r.wait_send()
        out_dma.wait()

    wq_half_t = pltpu.VMEM((Wq.shape[0], DHH), jnp.bfloat16)
    wo_half_t = pltpu.VMEM((DHH, Wo.shape[1]), jnp.bfloat16)
    kv_t = pltpu.VMEM((B_LOC, SQ, H_TOT * DH), jnp.bfloat16)
    return pl.pallas_call(
        body,
        out_shape=jax.ShapeDtypeStruct(x.shape, jnp.float32),
        in_specs=[
            pl.BlockSpec(memory_space=pltpu.VMEM),
            pl.BlockSpec(memory_space=pltpu.VMEM),
            pl.BlockSpec(memory_space=pl.ANY),
            pl.BlockSpec(memory_space=pl.ANY),
            pl.BlockSpec(memory_space=pltpu.VMEM),
        ],
        out_specs=pl.BlockSpec(memory_space=pl.ANY),
        scratch_shapes=[
            pltpu.VMEM(x.shape, jnp.bfloat16),
            pltpu.VMEM(x.shape, jnp.float32),
            pltpu.VMEM((B_LOC, SQ, H_TOT * DH), jnp.float32),
            pltpu.VMEM((B_LOC, SQ, H_TOT * DH), jnp.float32),
            wq_half_t, wq_half_t, wo_half_t, wo_half_t,
            wq_half_t, wo_half_t, wq_half_t, wo_half_t,
            wq_half_t, wo_half_t, wq_half_t, wo_half_t,
            wq_half_t, wo_half_t, wq_half_t, wo_half_t,
            kv_t, kv_t,
            pltpu.SemaphoreType.DMA((2,)),
            pltpu.SemaphoreType.DMA((12,)),
            pltpu.SemaphoreType.DMA((12,)),
        ],
        compiler_params=pltpu.CompilerParams(collective_id=0),
    )(x, Wq, K_r, V_r, Wo)


# device time: 20822 ns/iter; 1.1849x vs baseline; 1.1849x over previous
import jax
import jax.numpy as jnp
from jax import lax
from jax.experimental import pallas as pl
from jax.experimental.pallas import tpu as pltpu

N_DEV = 4
B_LOC = 2
H_TOT = 16
H_LOC = 4
H_HALF = 2
SQ = 128
DH = 64
DHH = H_HALF * DH
SCALE = 0.125


def kernel(x, Wq, K_ext, V_ext, Wo):
    my = lax.axis_index("i")
    K_my = lax.dynamic_slice_in_dim(K_ext, my * B_LOC, B_LOC, axis=0)
    V_my = lax.dynamic_slice_in_dim(V_ext, my * B_LOC, B_LOC, axis=0)
    K_t = K_my.astype(jnp.bfloat16).reshape(B_LOC, SQ, H_TOT * DH)
    V_t = V_my.astype(jnp.bfloat16).reshape(B_LOC, SQ, H_TOT * DH)

    def body(x_ref, wq_ref, k_any, v_any, wo_ref, out_hbm,
             x16, acc, snd_wq_cw, snd_wq_ccw, snd_wo_cw, snd_wo_ccw,
             l_wq_cw, l_wo_cw, l_wq_ccw, l_wo_ccw,
             r_wq_cw, r_wo_cw, r_wq_ccw, r_wo_ccw,
             o_wq_cw, o_wo_cw, o_wq_ccw, o_wo_ccw,
             kt, vt, raw_sems, send_sems, recv_sems):
        me = lax.axis_index("i")
        right = lax.rem(me + 1, N_DEV)
        left = lax.rem(me + N_DEV - 1, N_DEV)
        opp = lax.rem(me + 2, N_DEV)

        raw_dmas = [
            pltpu.make_async_copy(k_any, kt, raw_sems.at[0]),
            pltpu.make_async_copy(v_any, vt, raw_sems.at[1]),
        ]
        for d in raw_dmas:
            d.start()

        barrier_sem = pltpu.get_barrier_semaphore()
        for nbr in (left, right):
            pl.semaphore_signal(
                barrier_sem, inc=1,
                device_id=(nbr,), device_id_type=pl.DeviceIdType.MESH,
            )
        pl.semaphore_wait(barrier_sem, 2)

        snd_wq_cw[...] = wq_ref[:, :DHH].astype(jnp.bfloat16)
        snd_wq_ccw[...] = wq_ref[:, DHH:].astype(jnp.bfloat16)
        snd_wo_cw[...] = wo_ref[:DHH, :].astype(jnp.bfloat16)
        snd_wo_ccw[...] = wo_ref[DHH:, :].astype(jnp.bfloat16)

        def mk(src, dst, idx, nbr):
            return pltpu.make_async_remote_copy(
                src_ref=src, dst_ref=dst,
                send_sem=send_sems.at[idx], recv_sem=recv_sems.at[idx],
                device_id=(nbr,), device_id_type=pl.DeviceIdType.MESH,
            )

        hop1 = [
            mk(snd_wq_cw, l_wq_cw, 0, right),
            mk(snd_wo_cw, l_wo_cw, 1, right),
            mk(snd_wq_ccw, l_wq_ccw, 2, right),
            mk(snd_wo_ccw, l_wo_ccw, 3, right),
            mk(snd_wq_ccw, r_wq_ccw, 4, left),
            mk(snd_wo_ccw, r_wo_ccw, 5, left),
            mk(snd_wq_cw, r_wq_cw, 6, left),
            mk(snd_wo_cw, r_wo_cw, 7, left),
        ]
        for r in hop1:
            r.start()

        x16[...] = x_ref[...].astype(jnp.bfloat16)

        for d in raw_dmas:
            d.wait()

        def attend_half(b, q_half, o, head_off, wo_half):
            col0 = o * H_LOC * DH + head_off * DH
            k_pair = kt[b, :, pl.ds(col0, DHH)]
            v_pair = vt[b, :, pl.ds(col0, DHH)]
            ctx_parts = []
            for hh in range(H_HALF):
                k = k_pair[:, hh * DH:(hh + 1) * DH]
                v = v_pair[:, hh * DH:(hh + 1) * DH]
                q = q_half[:, hh * DH:(hh + 1) * DH]
                s = lax.dot_general(
                    q, k, (((1,), (1,)), ((), ())),
                    preferred_element_type=jnp.float32,
                ) * SCALE
                m = jnp.max(s, axis=-1, keepdims=True)
                w = jnp.exp(s - m)
                w = (w / jnp.sum(w, axis=-1, keepdims=True)).astype(
                    jnp.bfloat16
                )
                ctx_parts.append(
                    jnp.dot(w, v, preferred_element_type=jnp.float32)
                )
            ctx = jnp.concatenate(ctx_parts, axis=1).astype(jnp.bfloat16)
            return jnp.dot(ctx, wo_half, preferred_element_type=jnp.float32)

        def compute_half(origin, head_off, wq_half_ref, wo_half_ref,
                         init=False):
            wq_half = wq_half_ref[...]
            wo_half = wo_half_ref[...]
            for b in range(B_LOC):
                q = jnp.dot(
                    x16[b], wq_half, preferred_element_type=jnp.float32
                ).astype(jnp.bfloat16)
                c = attend_half(b, q, origin, head_off, wo_half)
                if init:
                    acc[b] = c
                else:
                    acc[b] = acc[b] + c

        compute_half(me, 0, snd_wq_cw, snd_wo_cw, init=True)
        compute_half(me, H_HALF, snd_wq_ccw, snd_wo_ccw)

        hop1[0].wait_recv()
        hop1[1].wait_recv()
        hop2 = [
            mk(l_wq_cw, o_wq_cw, 8, right),
            mk(l_wo_cw, o_wo_cw, 9, right),
        ]
        hop1[4].wait_recv()
        hop1[5].wait_recv()
        hop2 += [
            mk(r_wq_ccw, o_wq_ccw, 10, left),
            mk(r_wo_ccw, o_wo_ccw, 11, left),
        ]
        for r in hop2:
            r.start()

        compute_half(left, 0, l_wq_cw, l_wo_cw)
        compute_half(right, H_HALF, r_wq_ccw, r_wo_ccw)

        hop1[2].wait_recv()
        hop1[3].wait_recv()
        compute_half(left, H_HALF, l_wq_ccw, l_wo_ccw)

        hop1[6].wait_recv()
        hop1[7].wait_recv()
        compute_half(right, 0, r_wq_cw, r_wo_cw)

        hop2[0].wait_recv()
        hop2[1].wait_recv()
        compute_half(opp, 0, o_wq_cw, o_wo_cw)

        hop2[2].wait_recv()
        hop2[3].wait_recv()
        compute_half(opp, H_HALF, o_wq_ccw, o_wo_ccw)

        out_dma = pltpu.make_async_copy(acc, out_hbm, raw_sems.at[0])
        out_dma.start()

        for r in hop1 + hop2:
            r.wait_send()
        out_dma.wait()

    wq_half_t = pltpu.VMEM((Wq.shape[0], DHH), jnp.bfloat16)
    wo_half_t = pltpu.VMEM((DHH, Wo.shape[1]), jnp.bfloat16)
    kv_t = pltpu.VMEM((B_LOC, SQ, H_TOT * DH), jnp.bfloat16)
    return pl.pallas_call(
        body,
        out_shape=jax.ShapeDtypeStruct(x.shape, jnp.float32),
        in_specs=[
            pl.BlockSpec(memory_space=pltpu.VMEM),
            pl.BlockSpec(memory_space=pltpu.VMEM),
            pl.BlockSpec(memory_space=pl.ANY),
            pl.BlockSpec(memory_space=pl.ANY),
            pl.BlockSpec(memory_space=pltpu.VMEM),
        ],
        out_specs=pl.BlockSpec(memory_space=pl.ANY),
        scratch_shapes=[
            pltpu.VMEM(x.shape, jnp.bfloat16),
            pltpu.VMEM(x.shape, jnp.float32),
            wq_half_t, wq_half_t, wo_half_t, wo_half_t,
            wq_half_t, wo_half_t, wq_half_t, wo_half_t,
            wq_half_t, wo_half_t, wq_half_t, wo_half_t,
            wq_half_t, wo_half_t, wq_half_t, wo_half_t,
            kv_t, kv_t,
            pltpu.SemaphoreType.DMA((2,)),
            pltpu.SemaphoreType.DMA((12,)),
            pltpu.SemaphoreType.DMA((12,)),
        ],
        compiler_params=pltpu.CompilerParams(collective_id=0),
    )(x, Wq, K_t, V_t, Wo)


# device time: 19827 ns/iter; 1.2444x vs baseline; 1.0502x over previous
import jax
import jax.numpy as jnp
from jax import lax
from jax.experimental import pallas as pl
from jax.experimental.pallas import tpu as pltpu

N_DEV = 4
B_LOC = 2
H_TOT = 16
H_LOC = 4
H_HALF = 2
SQ = 128
DH = 64
DHH = H_HALF * DH
SCALE = 0.125


def kernel(x, Wq, K_ext, V_ext, Wo):
    my = lax.axis_index("i")
    K_my = lax.dynamic_slice_in_dim(K_ext, my * B_LOC, B_LOC, axis=0)
    V_my = lax.dynamic_slice_in_dim(V_ext, my * B_LOC, B_LOC, axis=0)
    K_t = K_my.astype(jnp.bfloat16).reshape(B_LOC, SQ, H_TOT * DH)
    V_t = V_my.astype(jnp.bfloat16).reshape(B_LOC, SQ, H_TOT * DH)

    def body(x_ref, wq_ref, k_any, v_any, wo_ref, out_hbm,
             x16, acc, snd_wq_cw, snd_wq_ccw, snd_wo_cw, snd_wo_ccw,
             l_wq_cw, l_wo_cw, l_wq_ccw, l_wo_ccw,
             r_wq_cw, r_wo_cw, r_wq_ccw, r_wo_ccw,
             o_wq_cw, o_wo_cw, o_wq_ccw, o_wo_ccw,
             kt, vt, raw_sems, send_sems, recv_sems):
        me = lax.axis_index("i")
        right = lax.rem(me + 1, N_DEV)
        left = lax.rem(me + N_DEV - 1, N_DEV)
        opp = lax.rem(me + 2, N_DEV)

        raw_dmas = [
            pltpu.make_async_copy(k_any, kt, raw_sems.at[0]),
            pltpu.make_async_copy(v_any, vt, raw_sems.at[1]),
        ]
        for d in raw_dmas:
            d.start()

        barrier_sem = pltpu.get_barrier_semaphore()
        for nbr in (left, right):
            pl.semaphore_signal(
                barrier_sem, inc=1,
                device_id=(nbr,), device_id_type=pl.DeviceIdType.MESH,
            )
        pl.semaphore_wait(barrier_sem, 2)

        snd_wq_cw[...] = wq_ref[:, :DHH].astype(jnp.bfloat16)
        snd_wq_ccw[...] = wq_ref[:, DHH:].astype(jnp.bfloat16)
        snd_wo_cw[...] = wo_ref[:DHH, :].astype(jnp.bfloat16)
        snd_wo_ccw[...] = wo_ref[DHH:, :].astype(jnp.bfloat16)

        def mk(src, dst, idx, nbr):
            return pltpu.make_async_remote_copy(
                src_ref=src, dst_ref=dst,
                send_sem=send_sems.at[idx], recv_sem=recv_sems.at[idx],
                device_id=(nbr,), device_id_type=pl.DeviceIdType.MESH,
            )

        hop1 = [
            mk(snd_wq_cw, l_wq_cw, 0, right),
            mk(snd_wo_cw, l_wo_cw, 1, right),
            mk(snd_wq_ccw, l_wq_ccw, 2, right),
            mk(snd_wo_ccw, l_wo_ccw, 3, right),
            mk(snd_wq_ccw, r_wq_ccw, 4, left),
            mk(snd_wo_ccw, r_wo_ccw, 5, left),
            mk(snd_wq_cw, r_wq_cw, 6, left),
            mk(snd_wo_cw, r_wo_cw, 7, left),
        ]
        for r in hop1:
            r.start()

        x16[...] = x_ref[...].astype(jnp.bfloat16)

        for d in raw_dmas:
            d.wait()

        def attend_half(b, q_half, o, head_off):
            col0 = o * H_LOC * DH + head_off * DH
            k_pair = kt[b, :, pl.ds(col0, DHH)]
            v_pair = vt[b, :, pl.ds(col0, DHH)]
            ctx_parts = []
            for hh in range(H_HALF):
                k = k_pair[:, hh * DH:(hh + 1) * DH]
                v = v_pair[:, hh * DH:(hh + 1) * DH]
                q = q_half[:, hh * DH:(hh + 1) * DH]
                s = lax.dot_general(
                    q, k, (((1,), (1,)), ((), ())),
                    preferred_element_type=jnp.float32,
                ) * SCALE
                m = jnp.max(s, axis=-1, keepdims=True)
                w = jnp.exp(s - m)
                w = (w / jnp.sum(w, axis=-1, keepdims=True)).astype(
                    jnp.bfloat16
                )
                ctx_parts.append(
                    jnp.dot(w, v, preferred_element_type=jnp.float32)
                )
            return jnp.concatenate(ctx_parts, axis=1).astype(jnp.bfloat16)

        def compute_half(origin, head_off, wq_half_ref, wo_half_ref,
                         init=False, wq_rdma=None, wo_rdma=None):
            if wq_rdma is not None:
                wq_rdma.wait_recv()
            wq_half = wq_half_ref[...]
            ctxs = []
            for b in range(B_LOC):
                q = jnp.dot(
                    x16[b], wq_half, preferred_element_type=jnp.float32
                ).astype(jnp.bfloat16)
                ctxs.append(attend_half(b, q, origin, head_off))
            if wo_rdma is not None:
                wo_rdma.wait_recv()
            wo_half = wo_half_ref[...]
            for b in range(B_LOC):
                c = jnp.dot(
                    ctxs[b], wo_half, preferred_element_type=jnp.float32
                )
                if init:
                    acc[b] = c
                else:
                    acc[b] = acc[b] + c

        compute_half(me, 0, snd_wq_cw, snd_wo_cw, init=True)
        compute_half(me, H_HALF, snd_wq_ccw, snd_wo_ccw)

        hop1[0].wait_recv()
        hop1[1].wait_recv()
        hop2 = [
            mk(l_wq_cw, o_wq_cw, 8, right),
            mk(l_wo_cw, o_wo_cw, 9, right),
        ]
        hop1[4].wait_recv()
        hop1[5].wait_recv()
        hop2 += [
            mk(r_wq_ccw, o_wq_ccw, 10, left),
            mk(r_wo_ccw, o_wo_ccw, 11, left),
        ]
        for r in hop2:
            r.start()

        compute_half(left, 0, l_wq_cw, l_wo_cw)
        compute_half(right, H_HALF, r_wq_ccw, r_wo_ccw)
        compute_half(left, H_HALF, l_wq_ccw, l_wo_ccw,
                     wq_rdma=hop1[2], wo_rdma=hop1[3])
        compute_half(right, 0, r_wq_cw, r_wo_cw,
                     wq_rdma=hop1[6], wo_rdma=hop1[7])
        compute_half(opp, 0, o_wq_cw, o_wo_cw,
                     wq_rdma=hop2[0], wo_rdma=hop2[1])
        compute_half(opp, H_HALF, o_wq_ccw, o_wo_ccw,
                     wq_rdma=hop2[2], wo_rdma=hop2[3])

        out_dma = pltpu.make_async_copy(acc, out_hbm, raw_sems.at[0])
        out_dma.start()

        for r in hop1 + hop2:
            r.wait_send()
        out_dma.wait()

    wq_half_t = pltpu.VMEM((Wq.shape[0], DHH), jnp.bfloat16)
    wo_half_t = pltpu.VMEM((DHH, Wo.shape[1]), jnp.bfloat16)
    kv_t = pltpu.VMEM((B_LOC, SQ, H_TOT * DH), jnp.bfloat16)
    return pl.pallas_call(
        body,
        out_shape=jax.ShapeDtypeStruct(x.shape, jnp.float32),
        in_specs=[
            pl.BlockSpec(memory_space=pltpu.VMEM),
            pl.BlockSpec(memory_space=pltpu.VMEM),
            pl.BlockSpec(memory_space=pl.ANY),
            pl.BlockSpec(memory_space=pl.ANY),
            pl.BlockSpec(memory_space=pltpu.VMEM),
        ],
        out_specs=pl.BlockSpec(memory_space=pl.ANY),
        scratch_shapes=[
            pltpu.VMEM(x.shape, jnp.bfloat16),
            pltpu.VMEM(x.shape, jnp.float32),
            wq_half_t, wq_half_t, wo_half_t, wo_half_t,
            wq_half_t, wo_half_t, wq_half_t, wo_half_t,
            wq_half_t, wo_half_t, wq_half_t, wo_half_t,
            wq_half_t, wo_half_t, wq_half_t, wo_half_t,
            kv_t, kv_t,
            pltpu.SemaphoreType.DMA((2,)),
            pltpu.SemaphoreType.DMA((12,)),
            pltpu.SemaphoreType.DMA((12,)),
        ],
        compiler_params=pltpu.CompilerParams(collective_id=0),
    )(x, Wq, K_t, V_t, Wo)


# device time: 19782 ns/iter; 1.2472x vs baseline; 1.0023x over previous
import jax
import jax.numpy as jnp
from jax import lax
from jax.experimental import pallas as pl
from jax.experimental.pallas import tpu as pltpu

N_DEV = 4
B_LOC = 2
H_TOT = 16
H_LOC = 4
H_HALF = 2
SQ = 128
DH = 64
DHH = H_HALF * DH
SCALE = 0.125


def kernel(x, Wq, K_ext, V_ext, Wo):
    my = lax.axis_index("i")
    K_my = lax.dynamic_slice_in_dim(K_ext, my * B_LOC, B_LOC, axis=0)
    V_my = lax.dynamic_slice_in_dim(V_ext, my * B_LOC, B_LOC, axis=0)
    K_t = K_my.astype(jnp.bfloat16).reshape(B_LOC, SQ, H_TOT * DH)
    V_t = V_my.astype(jnp.bfloat16).reshape(B_LOC, SQ, H_TOT * DH)

    def body(x_ref, wq_ref, k_any, v_any, wo_ref, out_hbm,
             x16, acc, snd_wq_cw, snd_wq_ccw, snd_wo_cw, snd_wo_ccw,
             l_wq_cw, l_wo_cw, l_wq_ccw, l_wo_ccw,
             r_wq_cw, r_wo_cw, r_wq_ccw, r_wo_ccw,
             o_wq_cw, o_wo_cw, o_wq_ccw, o_wo_ccw,
             kt, vt, raw_sems, send_sems, recv_sems):
        me = lax.axis_index("i")
        right = lax.rem(me + 1, N_DEV)
        left = lax.rem(me + N_DEV - 1, N_DEV)
        opp = lax.rem(me + 2, N_DEV)

        raw_dmas = [
            pltpu.make_async_copy(k_any, kt, raw_sems.at[0]),
            pltpu.make_async_copy(v_any, vt, raw_sems.at[1]),
        ]
        for d in raw_dmas:
            d.start()

        barrier_sem = pltpu.get_barrier_semaphore()
        for nbr in (left, right):
            pl.semaphore_signal(
                barrier_sem, inc=1,
                device_id=(nbr,), device_id_type=pl.DeviceIdType.MESH,
            )
        pl.semaphore_wait(barrier_sem, 2)

        snd_wq_cw[...] = wq_ref[:, :DHH].astype(jnp.bfloat16)
        snd_wo_cw[...] = wo_ref[:DHH, :].astype(jnp.bfloat16)

        def mk(src, dst, idx, nbr):
            return pltpu.make_async_remote_copy(
                src_ref=src, dst_ref=dst,
                send_sem=send_sems.at[idx], recv_sem=recv_sems.at[idx],
                device_id=(nbr,), device_id_type=pl.DeviceIdType.MESH,
            )

        hop1 = [
            mk(snd_wq_cw, l_wq_cw, 0, right),
            mk(snd_wo_cw, l_wo_cw, 1, right),
            mk(snd_wq_ccw, l_wq_ccw, 2, right),
            mk(snd_wo_ccw, l_wo_ccw, 3, right),
            mk(snd_wq_ccw, r_wq_ccw, 4, left),
            mk(snd_wo_ccw, r_wo_ccw, 5, left),
            mk(snd_wq_cw, r_wq_cw, 6, left),
            mk(snd_wo_cw, r_wo_cw, 7, left),
        ]
        hop1[0].start()
        hop1[1].start()
        snd_wq_ccw[...] = wq_ref[:, DHH:].astype(jnp.bfloat16)
        snd_wo_ccw[...] = wo_ref[DHH:, :].astype(jnp.bfloat16)
        for r in hop1[2:]:
            r.start()

        x16[...] = x_ref[...].astype(jnp.bfloat16)

        for d in raw_dmas:
            d.wait()

        def attend_half(b, q_half, o, head_off):
            col0 = o * H_LOC * DH + head_off * DH
            k_pair = kt[b, :, pl.ds(col0, DHH)]
            v_pair = vt[b, :, pl.ds(col0, DHH)]
            ctx_parts = []
            for hh in range(H_HALF):
                k = k_pair[:, hh * DH:(hh + 1) * DH]
                v = v_pair[:, hh * DH:(hh + 1) * DH]
                q = q_half[:, hh * DH:(hh + 1) * DH]
                s = lax.dot_general(
                    q, k, (((1,), (1,)), ((), ())),
                    preferred_element_type=jnp.float32,
                ) * SCALE
                m = jnp.max(s, axis=-1, keepdims=True)
                w = jnp.exp(s - m)
                w = (w / jnp.sum(w, axis=-1, keepdims=True)).astype(
                    jnp.bfloat16
                )
                ctx_parts.append(
                    jnp.dot(w, v, preferred_element_type=jnp.float32)
                )
            return jnp.concatenate(ctx_parts, axis=1).astype(jnp.bfloat16)

        def compute_half(origin, head_off, wq_half_ref, wo_half_ref,
                         init=False, wq_rdma=None, wo_rdma=None):
            if wq_rdma is not None:
                wq_rdma.wait_recv()
            wq_half = wq_half_ref[...]
            ctxs = []
            for b in range(B_LOC):
                q = jnp.dot(
                    x16[b], wq_half, preferred_element_type=jnp.float32
                ).astype(jnp.bfloat16)
                ctxs.append(attend_half(b, q, origin, head_off))
            if wo_rdma is not None:
                wo_rdma.wait_recv()
            wo_half = wo_half_ref[...]
            for b in range(B_LOC):
                c = jnp.dot(
                    ctxs[b], wo_half, preferred_element_type=jnp.float32
                )
                if init:
                    acc[b] = c
                else:
                    acc[b] = acc[b] + c

        compute_half(me, 0, snd_wq_cw, snd_wo_cw, init=True)
        compute_half(me, H_HALF, snd_wq_ccw, snd_wo_ccw)

        hop2 = [
            mk(l_wq_cw, o_wq_cw, 8, right),
            mk(l_wo_cw, o_wo_cw, 9, right),
            mk(r_wq_ccw, o_wq_ccw, 10, left),
            mk(r_wo_ccw, o_wo_ccw, 11, left),
        ]
        hop1[0].wait_recv()
        hop2[0].start()
        hop1[4].wait_recv()
        hop2[2].start()
        hop1[1].wait_recv()
        hop2[1].start()
        hop1[5].wait_recv()
        hop2[3].start()

        compute_half(left, 0, l_wq_cw, l_wo_cw)
        compute_half(right, H_HALF, r_wq_ccw, r_wo_ccw)
        compute_half(left, H_HALF, l_wq_ccw, l_wo_ccw,
                     wq_rdma=hop1[2], wo_rdma=hop1[3])
        compute_half(right, 0, r_wq_cw, r_wo_cw,
                     wq_rdma=hop1[6], wo_rdma=hop1[7])
        compute_half(opp, 0, o_wq_cw, o_wo_cw,
                     wq_rdma=hop2[0], wo_rdma=hop2[1])
        compute_half(opp, H_HALF, o_wq_ccw, o_wo_ccw,
                     wq_rdma=hop2[2], wo_rdma=hop2[3])

        out_dma = pltpu.make_async_copy(acc, out_hbm, raw_sems.at[0])
        out_dma.start()

        for r in hop1 + hop2:
            r.wait_send()
        out_dma.wait()

    wq_half_t = pltpu.VMEM((Wq.shape[0], DHH), jnp.bfloat16)
    wo_half_t = pltpu.VMEM((DHH, Wo.shape[1]), jnp.bfloat16)
    kv_t = pltpu.VMEM((B_LOC, SQ, H_TOT * DH), jnp.bfloat16)
    return pl.pallas_call(
        body,
        out_shape=jax.ShapeDtypeStruct(x.shape, jnp.float32),
        in_specs=[
            pl.BlockSpec(memory_space=pltpu.VMEM),
            pl.BlockSpec(memory_space=pltpu.VMEM),
            pl.BlockSpec(memory_space=pl.ANY),
            pl.BlockSpec(memory_space=pl.ANY),
            pl.BlockSpec(memory_space=pltpu.VMEM),
        ],
        out_specs=pl.BlockSpec(memory_space=pl.ANY),
        scratch_shapes=[
            pltpu.VMEM(x.shape, jnp.bfloat16),
            pltpu.VMEM(x.shape, jnp.float32),
            wq_half_t, wq_half_t, wo_half_t, wo_half_t,
            wq_half_t, wo_half_t, wq_half_t, wo_half_t,
            wq_half_t, wo_half_t, wq_half_t, wo_half_t,
            wq_half_t, wo_half_t, wq_half_t, wo_half_t,
            kv_t, kv_t,
            pltpu.SemaphoreType.DMA((2,)),
            pltpu.SemaphoreType.DMA((12,)),
            pltpu.SemaphoreType.DMA((12,)),
        ],
        compiler_params=pltpu.CompilerParams(collective_id=0),
    )(x, Wq, K_t, V_t, Wo)
